# baseline (speedup 1.0000x reference)
import math

import jax
import jax.numpy as jnp
from jax.experimental import pallas as pl
from jax.experimental.pallas import tpu as pltpu

B, EMB, N = 16, 128, 2048
N_SAMPLES = 15
TOPK = 16
BLK_R = 1024
SCALE = math.sqrt(float(EMB))


def _scores_stage_kernel(src_ref, tgt_ref, vals_ref, idx_ref, colsum_ref):
    rb = pl.program_id(1)
    s_blk = src_ref[0]
    t_all = tgt_ref[0]
    logits = jax.lax.dot_general(
        s_blk, t_all, (((0,), (0,)), ((), ())),
        preferred_element_type=jnp.float32,
    ) / SCALE

    m = jnp.max(logits, axis=-1, keepdims=True)
    e = jnp.exp(logits - m)
    s = jnp.sum(e, axis=-1, keepdims=True)

    part = jnp.sum(e * (1.0 / s), axis=0, keepdims=True)

    @pl.when(rb == 0)
    def _():
        colsum_ref[0] = part

    @pl.when(rb != 0)
    def _():
        colsum_ref[0] = colsum_ref[0] + part

    neg_iota = (-jax.lax.broadcasted_iota(jnp.int32, (BLK_R, N), 1)
                ).astype(jnp.float32)
    k_iota = jax.lax.broadcasted_iota(jnp.int32, (BLK_R, TOPK), 1)
    x = logits
    vals_acc = jnp.zeros((BLK_R, TOPK), jnp.float32)
    idx_acc = jnp.zeros((BLK_R, TOPK), jnp.float32)
    for k in range(TOPK):
        mk = jnp.max(x, axis=-1, keepdims=True)
        akf = jnp.max(jnp.where(x == mk, neg_iota, -jnp.inf),
                      axis=-1, keepdims=True)
        vals_acc = jnp.where(k_iota == k, jnp.exp(mk - m) / s, vals_acc)
        idx_acc = jnp.where(k_iota == k, akf, idx_acc)
        x = jnp.where(neg_iota == akf, -jnp.inf, x)
    vals_ref[0] = vals_acc
    idx_ref[0] = (-idx_acc).astype(jnp.int32)


def _match_stage_kernel(vals_ref, idx_ref, out_ref):
    v0 = jnp.transpose(vals_ref[0], (1, 0))
    colsf = jnp.transpose(idx_ref[0], (1, 0)).astype(jnp.float32)
    neg_lane = (-jax.lax.broadcasted_iota(jnp.int32, (1, N), 1)
                ).astype(jnp.float32)
    neg_sub = (-jax.lax.broadcasted_iota(jnp.int32, (TOPK, 1), 0)
               ).astype(jnp.float32)

    def body(i, carry):
        v, msel = carry
        rowbest = jnp.max(v, axis=0, keepdims=True)
        gmax = jnp.max(rowbest)
        rf = jnp.max(jnp.where(rowbest == gmax, neg_lane, -jnp.inf))
        rowmask = neg_lane == rf
        vrow = jnp.max(jnp.where(rowmask, v, -jnp.inf), axis=1,
                       keepdims=True)
        posf = jnp.max(jnp.where(vrow == gmax, neg_sub, -jnp.inf))
        cf = jnp.max(jnp.where(rowmask & (neg_sub == posf), colsf,
                               -jnp.inf))
        r = (-rf).astype(jnp.int32)
        c = cf.astype(jnp.int32)
        sel_mask = jax.lax.broadcasted_iota(jnp.int32, (TOPK, 2), 0) == i
        rc = jnp.concatenate(
            [jnp.full((TOPK, 1), r, jnp.int32),
             jnp.full((TOPK, 1), c, jnp.int32)], axis=1)
        msel = jnp.where(sel_mask, rc, msel)
        v = jnp.where(rowmask | (colsf == cf), -1.0, v)
        return v, msel

    _, msel = jax.lax.fori_loop(
        0, N_SAMPLES, body,
        (v0, jnp.zeros((TOPK, 2), jnp.int32)))
    out_ref[0] = msel


def _scores_stage(src_embedding, tgt_embedding):
    grid = (B, N // BLK_R)
    return pl.pallas_call(
        _scores_stage_kernel,
        grid=grid,
        in_specs=[
            pl.BlockSpec((1, EMB, BLK_R), lambda b, rb: (b, 0, rb)),
            pl.BlockSpec((1, EMB, N), lambda b, rb: (b, 0, 0)),
        ],
        out_specs=[
            pl.BlockSpec((1, BLK_R, TOPK), lambda b, rb: (b, rb, 0)),
            pl.BlockSpec((1, BLK_R, TOPK), lambda b, rb: (b, rb, 0)),
            pl.BlockSpec((1, 1, N), lambda b, rb: (b, 0, 0)),
        ],
        out_shape=[
            jax.ShapeDtypeStruct((B, N, TOPK), jnp.float32),
            jax.ShapeDtypeStruct((B, N, TOPK), jnp.int32),
            jax.ShapeDtypeStruct((B, 1, N), jnp.float32),
        ],
        compiler_params=pltpu.CompilerParams(
            dimension_semantics=("parallel", "arbitrary"),
        ),
    )(src_embedding, tgt_embedding)


def _match_stage(vals, idx):
    return pl.pallas_call(
        _match_stage_kernel,
        grid=(B,),
        in_specs=[
            pl.BlockSpec((1, N, TOPK), lambda b: (b, 0, 0)),
            pl.BlockSpec((1, N, TOPK), lambda b: (b, 0, 0)),
        ],
        out_specs=pl.BlockSpec((1, TOPK, 2), lambda b: (b, 0, 0)),
        out_shape=jax.ShapeDtypeStruct((B, TOPK, 2), jnp.int32),
    )(vals, idx)


def kernel(src_embedding, tgt_embedding, src, tgt):
    vals, idx, colsum = _scores_stage(src_embedding, tgt_embedding)
    samples = _match_stage(vals, idx)[:, :N_SAMPLES, :]

    src_p = jnp.transpose(src, (0, 2, 1))
    tgt_p = jnp.transpose(tgt, (0, 2, 1))

    topk_src = jnp.take_along_axis(
        src_p, samples[:, None, :, 0].astype(jnp.int32), axis=2)
    topk_tgt = jnp.take_along_axis(
        tgt_p, samples[:, None, :, 1].astype(jnp.int32), axis=2)

    reflect = jnp.diag(jnp.array([1.0, 1.0, -1.0], dtype=jnp.float32))

    def svd_one(ts, tt):
        tgt_centered = tt - tt.mean(axis=1, keepdims=True)
        src_centered = ts - ts.mean(axis=1, keepdims=True)
        H = jnp.matmul(src_centered, tgt_centered.T)
        u, _, vh = jnp.linalg.svd(H, full_matrices=False)
        v = vh.T
        r = jnp.matmul(v, u.T)
        v2 = jnp.where(jnp.linalg.det(r) < 0, jnp.matmul(v, reflect), v)
        return jnp.matmul(v2, u.T)

    R = jax.vmap(svd_one)(topk_src, topk_tgt)

    src_corr_mean = jnp.einsum(
        "bdn,bn->bd", tgt_p, colsum[:, 0, :]) / N
    t = (-jnp.einsum("bij,bj->bi", R, src_p.mean(axis=2))
         + src_corr_mean)
    return (R, t.reshape(B, 3))


# speedup vs baseline: 1.0106x; 1.0106x over previous
import math

import jax
import jax.numpy as jnp
from jax.experimental import pallas as pl
from jax.experimental.pallas import tpu as pltpu

B, EMB, N = 16, 128, 2048
N_SAMPLES = 15
TOPK = 16
BLK_R = 512
SCALE = math.sqrt(float(EMB))


def _scores_stage_kernel(src_ref, tgt_ref, vals_ref, idx_ref, colsum_ref):
    rb = pl.program_id(1)
    s_blk = src_ref[0]
    t_all = tgt_ref[0]
    logits = jax.lax.dot_general(
        s_blk, t_all, (((0,), (0,)), ((), ())),
        preferred_element_type=jnp.float32,
    ) / SCALE

    m = jnp.max(logits, axis=-1, keepdims=True)
    e = jnp.exp(logits - m)
    s = jnp.sum(e, axis=-1, keepdims=True)

    part = jnp.sum(e * (1.0 / s), axis=0, keepdims=True)

    @pl.when(rb == 0)
    def _():
        colsum_ref[0] = part

    @pl.when(rb != 0)
    def _():
        colsum_ref[0] = colsum_ref[0] + part

    neg_iota = (-jax.lax.broadcasted_iota(jnp.int32, (BLK_R, N), 1)
                ).astype(jnp.float32)
    k_iota = jax.lax.broadcasted_iota(jnp.int32, (BLK_R, TOPK), 1)
    x = logits
    vals_acc = jnp.zeros((BLK_R, TOPK), jnp.float32)
    idx_acc = jnp.zeros((BLK_R, TOPK), jnp.float32)
    for k in range(TOPK):
        mk = jnp.max(x, axis=-1, keepdims=True)
        akf = jnp.max(jnp.where(x == mk, neg_iota, -jnp.inf),
                      axis=-1, keepdims=True)
        vals_acc = jnp.where(k_iota == k, jnp.exp(mk - m) / s, vals_acc)
        idx_acc = jnp.where(k_iota == k, akf, idx_acc)
        x = jnp.where(neg_iota == akf, -jnp.inf, x)
    vals_ref[0] = vals_acc
    idx_ref[0] = (-idx_acc).astype(jnp.int32)


def _match_stage_kernel(vals_ref, idx_ref, src_ref, tgt_ref, colsum_ref,
                        out_ref, tsrc_ref, ttgt_ref, means_ref):
    v0 = jnp.transpose(vals_ref[0], (1, 0))
    colsf = jnp.transpose(idx_ref[0], (1, 0)).astype(jnp.float32)
    neg_lane = (-jax.lax.broadcasted_iota(jnp.int32, (1, N), 1)
                ).astype(jnp.float32)
    neg_sub = (-jax.lax.broadcasted_iota(jnp.int32, (TOPK, 1), 0)
               ).astype(jnp.float32)

    def body(i, carry):
        v, msel = carry
        rowbest = jnp.max(v, axis=0, keepdims=True)
        gmax = jnp.max(rowbest)
        rf = jnp.max(jnp.where(rowbest == gmax, neg_lane, -jnp.inf))
        rowmask = neg_lane == rf
        vrow = jnp.max(jnp.where(rowmask, v, -jnp.inf), axis=1,
                       keepdims=True)
        posf = jnp.max(jnp.where(vrow == gmax, neg_sub, -jnp.inf))
        cf = jnp.max(jnp.where(rowmask & (neg_sub == posf), colsf,
                               -jnp.inf))
        r = (-rf).astype(jnp.int32)
        c = cf.astype(jnp.int32)
        sel_mask = jax.lax.broadcasted_iota(jnp.int32, (TOPK, 2), 0) == i
        rc = jnp.concatenate(
            [jnp.full((TOPK, 1), r, jnp.int32),
             jnp.full((TOPK, 1), c, jnp.int32)], axis=1)
        msel = jnp.where(sel_mask, rc, msel)
        v = jnp.where(rowmask | (colsf == cf), -1.0, v)
        return v, msel

    _, msel = jax.lax.fori_loop(
        0, N_SAMPLES, body,
        (v0, jnp.zeros((TOPK, 2), jnp.int32)))
    out_ref[0] = msel

    src_all = src_ref[0]
    tgt_all = tgt_ref[0]
    iota16n = jax.lax.broadcasted_iota(jnp.int32, (TOPK, N), 1)
    oh_r = jnp.where(iota16n == msel[:, 0:1], 1.0, 0.0)
    oh_c = jnp.where(iota16n == msel[:, 1:2], 1.0, 0.0)
    dn = (((1,), (0,)), ((), ()))
    tsrc_ref[0] = jax.lax.dot_general(
        oh_r, src_all, dn, preferred_element_type=jnp.float32)
    ttgt_ref[0] = jax.lax.dot_general(
        oh_c, tgt_all, dn, preferred_element_type=jnp.float32)
    ones_row = jnp.full((1, N), 1.0 / N, jnp.float32)
    src_mean = jax.lax.dot_general(
        ones_row, src_all, dn, preferred_element_type=jnp.float32)
    corr_mean = jax.lax.dot_general(
        colsum_ref[0] * (1.0 / N), tgt_all, dn,
        preferred_element_type=jnp.float32)
    means_ref[0] = jnp.concatenate([src_mean, corr_mean], axis=0)


def _scores_stage(src_embedding, tgt_embedding):
    grid = (B, N // BLK_R)
    return pl.pallas_call(
        _scores_stage_kernel,
        grid=grid,
        in_specs=[
            pl.BlockSpec((1, EMB, BLK_R), lambda b, rb: (b, 0, rb)),
            pl.BlockSpec((1, EMB, N), lambda b, rb: (b, 0, 0)),
        ],
        out_specs=[
            pl.BlockSpec((1, BLK_R, TOPK), lambda b, rb: (b, rb, 0)),
            pl.BlockSpec((1, BLK_R, TOPK), lambda b, rb: (b, rb, 0)),
            pl.BlockSpec((1, 1, N), lambda b, rb: (b, 0, 0)),
        ],
        out_shape=[
            jax.ShapeDtypeStruct((B, N, TOPK), jnp.float32),
            jax.ShapeDtypeStruct((B, N, TOPK), jnp.int32),
            jax.ShapeDtypeStruct((B, 1, N), jnp.float32),
        ],
        compiler_params=pltpu.CompilerParams(
            dimension_semantics=("parallel", "arbitrary"),
        ),
    )(src_embedding, tgt_embedding)


def _match_stage(vals, idx, src, tgt, colsum):
    return pl.pallas_call(
        _match_stage_kernel,
        grid=(B,),
        in_specs=[
            pl.BlockSpec((1, N, TOPK), lambda b: (b, 0, 0)),
            pl.BlockSpec((1, N, TOPK), lambda b: (b, 0, 0)),
            pl.BlockSpec((1, N, 3), lambda b: (b, 0, 0)),
            pl.BlockSpec((1, N, 3), lambda b: (b, 0, 0)),
            pl.BlockSpec((1, 1, N), lambda b: (b, 0, 0)),
        ],
        out_specs=[
            pl.BlockSpec((1, TOPK, 2), lambda b: (b, 0, 0)),
            pl.BlockSpec((1, TOPK, 3), lambda b: (b, 0, 0)),
            pl.BlockSpec((1, TOPK, 3), lambda b: (b, 0, 0)),
            pl.BlockSpec((1, 2, 3), lambda b: (b, 0, 0)),
        ],
        out_shape=[
            jax.ShapeDtypeStruct((B, TOPK, 2), jnp.int32),
            jax.ShapeDtypeStruct((B, TOPK, 3), jnp.float32),
            jax.ShapeDtypeStruct((B, TOPK, 3), jnp.float32),
            jax.ShapeDtypeStruct((B, 2, 3), jnp.float32),
        ],
    )(vals, idx, src, tgt, colsum)


def kernel(src_embedding, tgt_embedding, src, tgt):
    vals, idx, colsum = _scores_stage(src_embedding, tgt_embedding)
    _, tsrc, ttgt, means = _match_stage(vals, idx, src, tgt, colsum)
    topk_src = jnp.transpose(tsrc[:, :N_SAMPLES, :], (0, 2, 1))
    topk_tgt = jnp.transpose(ttgt[:, :N_SAMPLES, :], (0, 2, 1))

    reflect = jnp.diag(jnp.array([1.0, 1.0, -1.0], dtype=jnp.float32))

    def svd_one(ts, tt):
        tgt_centered = tt - tt.mean(axis=1, keepdims=True)
        src_centered = ts - ts.mean(axis=1, keepdims=True)
        H = jnp.matmul(src_centered, tgt_centered.T)
        u, _, vh = jnp.linalg.svd(H, full_matrices=False)
        v = vh.T
        r = jnp.matmul(v, u.T)
        v2 = jnp.where(jnp.linalg.det(r) < 0, jnp.matmul(v, reflect), v)
        return jnp.matmul(v2, u.T)

    R = jax.vmap(svd_one)(topk_src, topk_tgt)

    t = (-jnp.einsum("bij,bj->bi", R, means[:, 0, :]) + means[:, 1, :])
    return (R, t.reshape(B, 3))
